# baseline (speedup 1.0000x reference)
import functools

import jax
import jax.numpy as jnp
from jax import lax
from jax.experimental import pallas as pl
from jax.experimental.pallas import tpu as pltpu
from jax.experimental.pallas import tpu_sc as plsc

_N = 10000
_E = 320000
_IN_DIM = 128
_H = 64
_NGRAPH = 128
_NCLS = 10

_NC = 2
_NS = 16
_L = 16
_NW = _NC * _NS

_CH = 80
_NCHUNK = 128
_EPW = _CH * _NCHUNK
_EPAD = _NW * _EPW
_NPAD = 10240
_RPT = _N // _NS
_ZPT = _NPAD // _NS
_ZB = 40
_NBUF = 8
_G = 6
_SLOW = 1
_GAT_NF, _GAT_NS = 144, 112
_AGG_NF, _AGG_NS = 200, 56

_mesh = plsc.VectorSubcoreMesh(
    core_axis_name="c", subcore_axis_name="s", num_cores=_NC, num_subcores=_NS)


def _zero_fill(buf, words_div16):
  z = jnp.zeros((_L,), jnp.float32)

  @pl.loop(0, words_div16)
  def _(i):
    buf[pl.ds(i * _L, _L)] = z


def _zero_fill2d(buf, nrows, ncols):
  z = jnp.zeros((_L,), jnp.float32)

  @pl.loop(0, nrows)
  def _(r):
    for q in range(ncols // _L):
      buf[r, pl.ds(q * _L, _L)] = z


def _sc_gat_body(src3, dst3, asrc_hbm, adst_hbm, xl_hbm, part_hbm, esum_hbm,
                 *scr):
  nck = jnp.where(lax.axis_index("c") == _SLOW, _GAT_NS, _GAT_NF)
  sidx_v, didx_v, asrc_v, adst_v, zbuf_v, zvec_v = scr[:6]
  bufs = scr[6:6 + _NBUF]
  exb = scr[6 + _NBUF:6 + 2 * _NBUF]
  acc_sh, esum_sh = scr[6 + 2 * _NBUF:8 + 2 * _NBUF]
  sems = scr[8 + 2 * _NBUF:]
  gsems = sems[:_NBUF]
  ssems = sems[_NBUF:2 * _NBUF]
  esems = sems[2 * _NBUF:3 * _NBUF]

  c = lax.axis_index("c")
  s = lax.axis_index("s")
  wid = s * _NC + c

  _zero_fill2d(zbuf_v, _ZB, _H)
  _zero_fill(zvec_v, _ZPT // _L)
  for i in range(_ZPT // _ZB):
    pltpu.sync_copy(zbuf_v, acc_sh.at[pl.ds(s * _ZPT + i * _ZB, _ZB)])
  pltpu.sync_copy(zvec_v, esum_sh.at[pl.ds(s * _ZPT, _ZPT)])
  pltpu.sync_copy(src3.at[wid], sidx_v)
  pltpu.sync_copy(dst3.at[wid], didx_v)
  pltpu.sync_copy(asrc_hbm, asrc_v)
  pltpu.sync_copy(adst_hbm, adst_v)

  plsc.subcore_barrier()

  for i in range(_G):
    pltpu.async_copy(xl_hbm.at[sidx_v.at[i]], bufs[i], gsems[i])

  @pl.loop(0, nck, step=_NBUF)
  def _(ch):
    for b in range(_NBUF):
      cur = ch + b
      rb, eb = bufs[b], exb[b]
      nb = (b + _G) % _NBUF
      pltpu.make_async_copy(xl_hbm.at[sidx_v.at[cur]], rb, gsems[b]).wait()

      @pl.when(cur >= _NBUF - _G)
      def _():
        pltpu.make_async_copy(
            bufs[nb], acc_sh.at[didx_v.at[cur + _G - _NBUF]], ssems[nb]).wait()
        pltpu.make_async_copy(
            exb[nb], esum_sh.at[didx_v.at[cur + _G - _NBUF]], esems[nb]).wait()

      @pl.when(cur + _G < nck)
      def _():
        pltpu.async_copy(xl_hbm.at[sidx_v.at[cur + _G]], bufs[nb], gsems[nb])

      for j in range(_CH // _L):
        si = sidx_v[cur, pl.ds(j * _L, _L)]
        di = didx_v[cur, pl.ds(j * _L, _L)]
        e = plsc.load_gather(asrc_v, [si]) + plsc.load_gather(adst_v, [di])
        e = jnp.where(e > 0, e, 0.2 * e)
        eb[pl.ds(j * _L, _L)] = jnp.exp(e)

      @pl.loop(0, _CH // _L)
      def _(j):
        ex16 = eb[pl.ds(j * _L, _L)]
        for k in range(_L):
          r = j * _L + k
          w = ex16[k]
          for q in range(_H // _L):
            rb[r, pl.ds(q * _L, _L)] = rb[r, pl.ds(q * _L, _L)] * w

      pltpu.async_copy(rb, acc_sh.at[didx_v.at[cur]], ssems[b], add=True)
      pltpu.async_copy(eb, esum_sh.at[didx_v.at[cur]], esems[b], add=True)

  for dj in range(_NBUF - _G, 0, -1):
    bj = (_NBUF - dj) % _NBUF
    pltpu.make_async_copy(bufs[bj], acc_sh.at[didx_v.at[nck - dj]], ssems[bj]).wait()
    pltpu.make_async_copy(exb[bj], esum_sh.at[didx_v.at[nck - dj]], esems[bj]).wait()

  plsc.subcore_barrier()

  pltpu.sync_copy(acc_sh.at[pl.ds(s * _ZPT, _ZPT)],
                  part_hbm.at[c, pl.ds(s * _ZPT, _ZPT)])
  pltpu.sync_copy(esum_sh.at[pl.ds(s * _ZPT, _ZPT)],
                  esum_hbm.at[c, pl.ds(s * _ZPT, _ZPT)])


_sc_gat = functools.partial(
    pl.kernel,
    out_type=(
        jax.ShapeDtypeStruct((_NC, _NPAD, _H), jnp.float32),
        jax.ShapeDtypeStruct((_NC, _NPAD), jnp.float32),
    ),
    mesh=_mesh,
    compiler_params=pltpu.CompilerParams(
        needs_layout_passes=False, use_tc_tiling_on_sc=False),
    scratch_types=[
        pltpu.VMEM((_GAT_NF, _CH), jnp.int32),
        pltpu.VMEM((_GAT_NF, _CH), jnp.int32),
        pltpu.VMEM((_N,), jnp.float32),
        pltpu.VMEM((_N,), jnp.float32),
        pltpu.VMEM((_ZB, _H), jnp.float32),
        pltpu.VMEM((_ZPT,), jnp.float32),
    ] + [pltpu.VMEM((_CH, _H), jnp.float32) for _ in range(_NBUF)]
    + [pltpu.VMEM((_CH,), jnp.float32) for _ in range(_NBUF)]
    + [
        pltpu.VMEM_SHARED((_NPAD, _H), jnp.float32),
        pltpu.VMEM_SHARED((_NPAD,), jnp.float32),
    ] + [pltpu.SemaphoreType.DMA for _ in range(3 * _NBUF)],
)(_sc_gat_body)


def _sc_agg_body(src3, dst3, h_hbm, part_hbm, *scr):
  nck = jnp.where(lax.axis_index("c") == _SLOW, _AGG_NS, _AGG_NF)
  sidx_v, didx_v, zbuf_v = scr[:3]
  bufs = scr[3:3 + _NBUF]
  acc_sh = scr[3 + _NBUF]
  sems = scr[4 + _NBUF:]
  gsems = sems[:_NBUF]
  ssems = sems[_NBUF:2 * _NBUF]

  c = lax.axis_index("c")
  s = lax.axis_index("s")
  wid = s * _NC + c

  _zero_fill2d(zbuf_v, _ZB, _H)
  for i in range(_ZPT // _ZB):
    pltpu.sync_copy(zbuf_v, acc_sh.at[pl.ds(s * _ZPT + i * _ZB, _ZB)])
  pltpu.sync_copy(src3.at[wid], sidx_v)
  pltpu.sync_copy(dst3.at[wid], didx_v)

  plsc.subcore_barrier()

  for i in range(_G):
    pltpu.async_copy(h_hbm.at[sidx_v.at[i]], bufs[i], gsems[i])

  @pl.loop(0, nck, step=_NBUF)
  def _(ch):
    for b in range(_NBUF):
      cur = ch + b
      rb = bufs[b]
      nb = (b + _G) % _NBUF
      pltpu.make_async_copy(h_hbm.at[sidx_v.at[cur]], rb, gsems[b]).wait()

      @pl.when(cur >= _NBUF - _G)
      def _():
        pltpu.make_async_copy(
            bufs[nb], acc_sh.at[didx_v.at[cur + _G - _NBUF]], ssems[nb]).wait()

      @pl.when(cur + _G < nck)
      def _():
        pltpu.async_copy(h_hbm.at[sidx_v.at[cur + _G]], bufs[nb], gsems[nb])

      pltpu.async_copy(rb, acc_sh.at[didx_v.at[cur]], ssems[b], add=True)

  for dj in range(_NBUF - _G, 0, -1):
    bj = (_NBUF - dj) % _NBUF
    pltpu.make_async_copy(bufs[bj], acc_sh.at[didx_v.at[nck - dj]], ssems[bj]).wait()

  plsc.subcore_barrier()

  pltpu.sync_copy(acc_sh.at[pl.ds(s * _ZPT, _ZPT)],
                  part_hbm.at[c, pl.ds(s * _ZPT, _ZPT)])


_sc_agg = functools.partial(
    pl.kernel,
    out_type=jax.ShapeDtypeStruct((_NC, _NPAD, _H), jnp.float32),
    mesh=_mesh,
    compiler_params=pltpu.CompilerParams(
        needs_layout_passes=False, use_tc_tiling_on_sc=False),
    scratch_types=[
        pltpu.VMEM((_AGG_NF, _CH), jnp.int32),
        pltpu.VMEM((_AGG_NF, _CH), jnp.int32),
        pltpu.VMEM((_ZB, _H), jnp.float32),
    ] + [pltpu.VMEM((_CH, _H), jnp.float32) for _ in range(_NBUF)]
    + [pltpu.VMEM_SHARED((_NPAD, _H), jnp.float32)]
    + [pltpu.SemaphoreType.DMA for _ in range(2 * _NBUF)],
)(_sc_agg_body)


def _bn_relu(z, g, b):
  m = jnp.mean(z, axis=0)
  v = jnp.mean((z - m) ** 2, axis=0)
  return jnp.maximum((z - m) / jnp.sqrt(v + 1e-5) * g + b, 0.0)


def _tc_pre_body(x_ref, w_ref, as_ref, ad_ref, xl_ref, a2_ref):
  xl = jnp.dot(x_ref[...], w_ref[...], preferred_element_type=jnp.float32)
  xl_ref[...] = xl
  asrc = jnp.sum(xl * as_ref[...][None, :], axis=1)
  adst = jnp.sum(xl * ad_ref[...][None, :], axis=1)
  a2_ref[...] = jnp.concatenate([asrc[:, None], adst[:, None]], axis=1)


def _tc_gat_post_body(part_ref, esum_ref, gb_ref, g_ref, b_ref, out_ref):
  p = part_ref[0, :_N] + part_ref[1, :_N]
  es = esum_ref[0, :_N] + esum_ref[1, :_N]
  z = p / (es[:, None] + 1e-16) + gb_ref[...][None, :]
  out_ref[...] = _bn_relu(z, g_ref[...][None, :], b_ref[...][None, :])


def _tc_gin_body(h_ref, part_ref, w1_ref, b1_ref, ng_ref, nb_ref,
                 w2_ref, b2_ref, og_ref, ob_ref, out_ref):
  z = h_ref[...] + part_ref[0, :_N] + part_ref[1, :_N]
  t = jnp.dot(z, w1_ref[...], preferred_element_type=jnp.float32)
  t = _bn_relu(t + b1_ref[...][None, :], ng_ref[...][None, :],
               nb_ref[...][None, :])
  t = jnp.dot(t, w2_ref[...], preferred_element_type=jnp.float32)
  t = t + b2_ref[...][None, :]
  out_ref[...] = _bn_relu(t, og_ref[...][None, :], ob_ref[...][None, :])


def _tc_final_body(g1_ref, g2_ref, g3_ref, batch_ref,
                   f1w_ref, f1b_ref, f2w_ref, f2b_ref, cw_ref, cb_ref,
                   out_ref):
  gcn = jnp.concatenate([g1_ref[...], g2_ref[...], g3_ref[...]], axis=1)
  gid = lax.broadcasted_iota(jnp.int32, (_NGRAPH, 1), 0)
  onehot = (gid == batch_ref[...]).astype(jnp.float32)
  pooled = jnp.dot(onehot, gcn, preferred_element_type=jnp.float32)
  h = jnp.dot(pooled, f1w_ref[...], preferred_element_type=jnp.float32)
  h = h + f1b_ref[...][None, :]
  h = jnp.dot(h, f2w_ref[...], preferred_element_type=jnp.float32)
  h = jnp.maximum(h + f2b_ref[...][None, :], 0.0)
  o = jnp.dot(h, cw_ref[...], preferred_element_type=jnp.float32)
  out_ref[...] = o + cb_ref[...][None, :]


def kernel(x, edge, batch, gat_W, gat_att_src, gat_att_dst, gat_b, bn1_g,
           bn1_b, g2_W1, g2_b1, g2_ng, g2_nb, g2_W2, g2_b2, g2_og, g2_ob,
           g3_W1, g3_b1, g3_ng, g3_nb, g3_W2, g3_b2, g3_og, g3_ob,
           fc1_W, fc1_b, fc2_W, fc2_b, cls_W, cls_b):
  src = edge[0].astype(jnp.int32)
  dst = edge[1].astype(jnp.int32)
  pad = _EPAD - _E
  src_p = jnp.concatenate([src, jnp.zeros((pad,), jnp.int32)])
  dst_p = jnp.concatenate([dst, jnp.full((pad,), _N, jnp.int32)])

  def _layout(x, n_fast, n_slow, fill):
    cut = _NS * n_fast * _CH
    e_fast = x[:cut].reshape(_NS, n_fast, _CH)
    e_slow = jnp.pad(
        x[cut:].reshape(_NS, n_slow, _CH),
        ((0, 0), (0, n_fast - n_slow), (0, 0)), constant_values=fill)
    pair = (e_slow, e_fast) if _SLOW == 0 else (e_fast, e_slow)
    return jnp.stack(pair, axis=1).reshape(_NW, n_fast, _CH)

  src3g = _layout(src_p, _GAT_NF, _GAT_NS, 0)
  dst3g = _layout(dst_p, _GAT_NF, _GAT_NS, _N)
  src3a = _layout(src_p, _AGG_NF, _AGG_NS, 0)
  dst3a = _layout(dst_p, _AGG_NF, _AGG_NS, _N)
  batch_row = batch.astype(jnp.int32)[None, :]

  xl, a2 = pl.pallas_call(
      _tc_pre_body,
      out_shape=(
          jax.ShapeDtypeStruct((_N, _H), jnp.float32),
          jax.ShapeDtypeStruct((_N, 2), jnp.float32),
      ),
  )(x, gat_W, gat_att_src, gat_att_dst)
  asrc = a2[:, 0]
  adst = a2[:, 1]

  part1, esum = _sc_gat(src3g, dst3g, asrc, adst, xl)

  gcn1 = pl.pallas_call(
      _tc_gat_post_body,
      out_shape=jax.ShapeDtypeStruct((_N, _H), jnp.float32),
  )(part1, esum, gat_b, bn1_g, bn1_b)

  part2 = _sc_agg(src3a, dst3a, gcn1)
  gcn2 = pl.pallas_call(
      _tc_gin_body,
      out_shape=jax.ShapeDtypeStruct((_N, _H), jnp.float32),
  )(gcn1, part2, g2_W1, g2_b1, g2_ng, g2_nb, g2_W2, g2_b2, g2_og, g2_ob)

  part3 = _sc_agg(src3a, dst3a, gcn2)
  gcn3 = pl.pallas_call(
      _tc_gin_body,
      out_shape=jax.ShapeDtypeStruct((_N, _H), jnp.float32),
  )(gcn2, part3, g3_W1, g3_b1, g3_ng, g3_nb, g3_W2, g3_b2, g3_og, g3_ob)

  out = pl.pallas_call(
      _tc_final_body,
      out_shape=jax.ShapeDtypeStruct((_NGRAPH, _NCLS), jnp.float32),
  )(gcn1, gcn2, gcn3, batch_row, fc1_W, fc1_b, fc2_W, fc2_b, cls_W, cls_b)
  return out


# speedup vs baseline: 1.0540x; 1.0540x over previous
import functools

import jax
import jax.numpy as jnp
from jax import lax
from jax.experimental import pallas as pl
from jax.experimental.pallas import tpu as pltpu
from jax.experimental.pallas import tpu_sc as plsc

_N = 10000
_E = 320000
_IN_DIM = 128
_H = 64
_NGRAPH = 128
_NCLS = 10

_NC = 2
_NS = 16
_L = 16
_NW = _NC * _NS

_CH = 80
_NCHUNK = 128
_EPW = _CH * _NCHUNK
_EPAD = _NW * _EPW
_NPAD = 10240
_RPT = _N // _NS
_ZPT = _NPAD // _NS
_ZB = 40
_NBUF = 8
_G = 6
_SLOW = 0
_GAT_NF, _GAT_NS = 144, 112
_AGG_NF, _AGG_NS = 200, 56

_mesh = plsc.VectorSubcoreMesh(
    core_axis_name="c", subcore_axis_name="s", num_cores=_NC, num_subcores=_NS)


def _zero_fill(buf, words_div16):
  z = jnp.zeros((_L,), jnp.float32)

  @pl.loop(0, words_div16)
  def _(i):
    buf[pl.ds(i * _L, _L)] = z


def _zero_fill2d(buf, nrows, ncols):
  z = jnp.zeros((_L,), jnp.float32)

  @pl.loop(0, nrows)
  def _(r):
    for q in range(ncols // _L):
      buf[r, pl.ds(q * _L, _L)] = z


def _sc_gat_body(src3, dst3, asrc_hbm, adst_hbm, xl_hbm, part_hbm, esum_hbm,
                 *scr):
  nck = jnp.where(lax.axis_index("c") == _SLOW, _GAT_NS, _GAT_NF)
  sidx_v, didx_v, asrc_v, adst_v, zbuf_v, zvec_v = scr[:6]
  bufs = scr[6:6 + _NBUF]
  exb = scr[6 + _NBUF:6 + 2 * _NBUF]
  acc_sh, esum_sh = scr[6 + 2 * _NBUF:8 + 2 * _NBUF]
  sems = scr[8 + 2 * _NBUF:]
  gsems = sems[:_NBUF]
  ssems = sems[_NBUF:2 * _NBUF]
  esems = sems[2 * _NBUF:3 * _NBUF]

  c = lax.axis_index("c")
  s = lax.axis_index("s")
  wid = s * _NC + c

  _zero_fill2d(zbuf_v, _ZB, _H)
  _zero_fill(zvec_v, _ZPT // _L)
  for i in range(_ZPT // _ZB):
    pltpu.sync_copy(zbuf_v, acc_sh.at[pl.ds(s * _ZPT + i * _ZB, _ZB)])
  pltpu.sync_copy(zvec_v, esum_sh.at[pl.ds(s * _ZPT, _ZPT)])
  pltpu.sync_copy(src3.at[wid], sidx_v)
  pltpu.sync_copy(dst3.at[wid], didx_v)
  pltpu.sync_copy(asrc_hbm, asrc_v)
  pltpu.sync_copy(adst_hbm, adst_v)

  plsc.subcore_barrier()

  for i in range(_G):
    pltpu.async_copy(xl_hbm.at[sidx_v.at[i]], bufs[i], gsems[i])

  @pl.loop(0, nck, step=_NBUF)
  def _(ch):
    for b in range(_NBUF):
      cur = ch + b
      rb, eb = bufs[b], exb[b]
      nb = (b + _G) % _NBUF
      pltpu.make_async_copy(xl_hbm.at[sidx_v.at[cur]], rb, gsems[b]).wait()

      @pl.when(cur >= _NBUF - _G)
      def _():
        pltpu.make_async_copy(
            bufs[nb], acc_sh.at[didx_v.at[cur + _G - _NBUF]], ssems[nb]).wait()
        pltpu.make_async_copy(
            exb[nb], esum_sh.at[didx_v.at[cur + _G - _NBUF]], esems[nb]).wait()

      @pl.when(cur + _G < nck)
      def _():
        pltpu.async_copy(xl_hbm.at[sidx_v.at[cur + _G]], bufs[nb], gsems[nb])

      for j in range(_CH // _L):
        si = sidx_v[cur, pl.ds(j * _L, _L)]
        di = didx_v[cur, pl.ds(j * _L, _L)]
        e = plsc.load_gather(asrc_v, [si]) + plsc.load_gather(adst_v, [di])
        e = jnp.where(e > 0, e, 0.2 * e)
        eb[pl.ds(j * _L, _L)] = jnp.exp(e)

      @pl.loop(0, _CH // _L)
      def _(j):
        ex16 = eb[pl.ds(j * _L, _L)]
        for k in range(_L):
          r = j * _L + k
          w = ex16[k]
          for q in range(_H // _L):
            rb[r, pl.ds(q * _L, _L)] = rb[r, pl.ds(q * _L, _L)] * w

      pltpu.async_copy(rb, acc_sh.at[didx_v.at[cur]], ssems[b], add=True)
      pltpu.async_copy(eb, esum_sh.at[didx_v.at[cur]], esems[b], add=True)

  for dj in range(_NBUF - _G, 0, -1):
    bj = (_NBUF - dj) % _NBUF
    pltpu.make_async_copy(bufs[bj], acc_sh.at[didx_v.at[nck - dj]], ssems[bj]).wait()
    pltpu.make_async_copy(exb[bj], esum_sh.at[didx_v.at[nck - dj]], esems[bj]).wait()

  plsc.subcore_barrier()

  pltpu.sync_copy(acc_sh.at[pl.ds(s * _ZPT, _ZPT)],
                  part_hbm.at[c, pl.ds(s * _ZPT, _ZPT)])
  pltpu.sync_copy(esum_sh.at[pl.ds(s * _ZPT, _ZPT)],
                  esum_hbm.at[c, pl.ds(s * _ZPT, _ZPT)])


_sc_gat = functools.partial(
    pl.kernel,
    out_type=(
        jax.ShapeDtypeStruct((_NC, _NPAD, _H), jnp.float32),
        jax.ShapeDtypeStruct((_NC, _NPAD), jnp.float32),
    ),
    mesh=_mesh,
    compiler_params=pltpu.CompilerParams(
        needs_layout_passes=False, use_tc_tiling_on_sc=False),
    scratch_types=[
        pltpu.VMEM((_GAT_NF, _CH), jnp.int32),
        pltpu.VMEM((_GAT_NF, _CH), jnp.int32),
        pltpu.VMEM((_N,), jnp.float32),
        pltpu.VMEM((_N,), jnp.float32),
        pltpu.VMEM((_ZB, _H), jnp.float32),
        pltpu.VMEM((_ZPT,), jnp.float32),
    ] + [pltpu.VMEM((_CH, _H), jnp.float32) for _ in range(_NBUF)]
    + [pltpu.VMEM((_CH,), jnp.float32) for _ in range(_NBUF)]
    + [
        pltpu.VMEM_SHARED((_NPAD, _H), jnp.float32),
        pltpu.VMEM_SHARED((_NPAD,), jnp.float32),
    ] + [pltpu.SemaphoreType.DMA for _ in range(3 * _NBUF)],
)(_sc_gat_body)


def _sc_agg_body(src3, dst3, h_hbm, part_hbm, *scr):
  nck = jnp.where(lax.axis_index("c") == _SLOW, _AGG_NS, _AGG_NF)
  sidx_v, didx_v, zbuf_v = scr[:3]
  bufs = scr[3:3 + _NBUF]
  acc_sh = scr[3 + _NBUF]
  sems = scr[4 + _NBUF:]
  gsems = sems[:_NBUF]
  ssems = sems[_NBUF:2 * _NBUF]

  c = lax.axis_index("c")
  s = lax.axis_index("s")
  wid = s * _NC + c

  _zero_fill2d(zbuf_v, _ZB, _H)
  for i in range(_ZPT // _ZB):
    pltpu.sync_copy(zbuf_v, acc_sh.at[pl.ds(s * _ZPT + i * _ZB, _ZB)])
  pltpu.sync_copy(src3.at[wid], sidx_v)
  pltpu.sync_copy(dst3.at[wid], didx_v)

  plsc.subcore_barrier()

  for i in range(_G):
    pltpu.async_copy(h_hbm.at[sidx_v.at[i]], bufs[i], gsems[i])

  @pl.loop(0, nck, step=_NBUF)
  def _(ch):
    for b in range(_NBUF):
      cur = ch + b
      rb = bufs[b]
      nb = (b + _G) % _NBUF
      pltpu.make_async_copy(h_hbm.at[sidx_v.at[cur]], rb, gsems[b]).wait()

      @pl.when(cur >= _NBUF - _G)
      def _():
        pltpu.make_async_copy(
            bufs[nb], acc_sh.at[didx_v.at[cur + _G - _NBUF]], ssems[nb]).wait()

      @pl.when(cur + _G < nck)
      def _():
        pltpu.async_copy(h_hbm.at[sidx_v.at[cur + _G]], bufs[nb], gsems[nb])

      pltpu.async_copy(rb, acc_sh.at[didx_v.at[cur]], ssems[b], add=True)

  for dj in range(_NBUF - _G, 0, -1):
    bj = (_NBUF - dj) % _NBUF
    pltpu.make_async_copy(bufs[bj], acc_sh.at[didx_v.at[nck - dj]], ssems[bj]).wait()

  plsc.subcore_barrier()

  pltpu.sync_copy(acc_sh.at[pl.ds(s * _ZPT, _ZPT)],
                  part_hbm.at[c, pl.ds(s * _ZPT, _ZPT)])


_sc_agg = functools.partial(
    pl.kernel,
    out_type=jax.ShapeDtypeStruct((_NC, _NPAD, _H), jnp.float32),
    mesh=_mesh,
    compiler_params=pltpu.CompilerParams(
        needs_layout_passes=False, use_tc_tiling_on_sc=False),
    scratch_types=[
        pltpu.VMEM((_AGG_NF, _CH), jnp.int32),
        pltpu.VMEM((_AGG_NF, _CH), jnp.int32),
        pltpu.VMEM((_ZB, _H), jnp.float32),
    ] + [pltpu.VMEM((_CH, _H), jnp.float32) for _ in range(_NBUF)]
    + [pltpu.VMEM_SHARED((_NPAD, _H), jnp.float32)]
    + [pltpu.SemaphoreType.DMA for _ in range(2 * _NBUF)],
)(_sc_agg_body)


def _bn_relu(z, g, b):
  m = jnp.mean(z, axis=0)
  v = jnp.mean((z - m) ** 2, axis=0)
  return jnp.maximum((z - m) / jnp.sqrt(v + 1e-5) * g + b, 0.0)


def _tc_pre_body(x_ref, w_ref, as_ref, ad_ref, xl_ref, a2_ref):
  xl = jnp.dot(x_ref[...], w_ref[...], preferred_element_type=jnp.float32)
  xl_ref[...] = xl
  asrc = jnp.sum(xl * as_ref[...][None, :], axis=1)
  adst = jnp.sum(xl * ad_ref[...][None, :], axis=1)
  a2_ref[...] = jnp.concatenate([asrc[:, None], adst[:, None]], axis=1)


def _tc_gat_post_body(part_ref, esum_ref, gb_ref, g_ref, b_ref, out_ref):
  p = part_ref[0, :_N] + part_ref[1, :_N]
  es = esum_ref[0, :_N] + esum_ref[1, :_N]
  z = p / (es[:, None] + 1e-16) + gb_ref[...][None, :]
  out_ref[...] = _bn_relu(z, g_ref[...][None, :], b_ref[...][None, :])


def _tc_gin_body(h_ref, part_ref, w1_ref, b1_ref, ng_ref, nb_ref,
                 w2_ref, b2_ref, og_ref, ob_ref, out_ref):
  z = h_ref[...] + part_ref[0, :_N] + part_ref[1, :_N]
  t = jnp.dot(z, w1_ref[...], preferred_element_type=jnp.float32)
  t = _bn_relu(t + b1_ref[...][None, :], ng_ref[...][None, :],
               nb_ref[...][None, :])
  t = jnp.dot(t, w2_ref[...], preferred_element_type=jnp.float32)
  t = t + b2_ref[...][None, :]
  out_ref[...] = _bn_relu(t, og_ref[...][None, :], ob_ref[...][None, :])


def _tc_final_body(g1_ref, g2_ref, g3_ref, batch_ref,
                   f1w_ref, f1b_ref, f2w_ref, f2b_ref, cw_ref, cb_ref,
                   out_ref):
  gcn = jnp.concatenate([g1_ref[...], g2_ref[...], g3_ref[...]], axis=1)
  gid = lax.broadcasted_iota(jnp.int32, (_NGRAPH, 1), 0)
  onehot = (gid == batch_ref[...]).astype(jnp.float32)
  pooled = jnp.dot(onehot, gcn, preferred_element_type=jnp.float32)
  h = jnp.dot(pooled, f1w_ref[...], preferred_element_type=jnp.float32)
  h = h + f1b_ref[...][None, :]
  h = jnp.dot(h, f2w_ref[...], preferred_element_type=jnp.float32)
  h = jnp.maximum(h + f2b_ref[...][None, :], 0.0)
  o = jnp.dot(h, cw_ref[...], preferred_element_type=jnp.float32)
  out_ref[...] = o + cb_ref[...][None, :]


def kernel(x, edge, batch, gat_W, gat_att_src, gat_att_dst, gat_b, bn1_g,
           bn1_b, g2_W1, g2_b1, g2_ng, g2_nb, g2_W2, g2_b2, g2_og, g2_ob,
           g3_W1, g3_b1, g3_ng, g3_nb, g3_W2, g3_b2, g3_og, g3_ob,
           fc1_W, fc1_b, fc2_W, fc2_b, cls_W, cls_b):
  src = edge[0].astype(jnp.int32)
  dst = edge[1].astype(jnp.int32)
  pad = _EPAD - _E
  src_p = jnp.concatenate([src, jnp.zeros((pad,), jnp.int32)])
  dst_p = jnp.concatenate([dst, jnp.full((pad,), _N, jnp.int32)])

  def _layout(x, n_fast, n_slow, fill):
    cut = _NS * n_fast * _CH
    e_fast = x[:cut].reshape(_NS, n_fast, _CH)
    e_slow = jnp.pad(
        x[cut:].reshape(_NS, n_slow, _CH),
        ((0, 0), (0, n_fast - n_slow), (0, 0)), constant_values=fill)
    pair = (e_slow, e_fast) if _SLOW == 0 else (e_fast, e_slow)
    return jnp.stack(pair, axis=1).reshape(_NW, n_fast, _CH)

  src3g = _layout(src_p, _GAT_NF, _GAT_NS, 0)
  dst3g = _layout(dst_p, _GAT_NF, _GAT_NS, _N)
  src3a = _layout(src_p, _AGG_NF, _AGG_NS, 0)
  dst3a = _layout(dst_p, _AGG_NF, _AGG_NS, _N)
  batch_row = batch.astype(jnp.int32)[None, :]

  xl, a2 = pl.pallas_call(
      _tc_pre_body,
      out_shape=(
          jax.ShapeDtypeStruct((_N, _H), jnp.float32),
          jax.ShapeDtypeStruct((_N, 2), jnp.float32),
      ),
  )(x, gat_W, gat_att_src, gat_att_dst)
  asrc = a2[:, 0]
  adst = a2[:, 1]

  part1, esum = _sc_gat(src3g, dst3g, asrc, adst, xl)

  gcn1 = pl.pallas_call(
      _tc_gat_post_body,
      out_shape=jax.ShapeDtypeStruct((_N, _H), jnp.float32),
  )(part1, esum, gat_b, bn1_g, bn1_b)

  part2 = _sc_agg(src3a, dst3a, gcn1)
  gcn2 = pl.pallas_call(
      _tc_gin_body,
      out_shape=jax.ShapeDtypeStruct((_N, _H), jnp.float32),
  )(gcn1, part2, g2_W1, g2_b1, g2_ng, g2_nb, g2_W2, g2_b2, g2_og, g2_ob)

  part3 = _sc_agg(src3a, dst3a, gcn2)
  gcn3 = pl.pallas_call(
      _tc_gin_body,
      out_shape=jax.ShapeDtypeStruct((_N, _H), jnp.float32),
  )(gcn2, part3, g3_W1, g3_b1, g3_ng, g3_nb, g3_W2, g3_b2, g3_og, g3_ob)

  out = pl.pallas_call(
      _tc_final_body,
      out_shape=jax.ShapeDtypeStruct((_NGRAPH, _NCLS), jnp.float32),
  )(gcn1, gcn2, gcn3, batch_row, fc1_W, fc1_b, fc2_W, fc2_b, cls_W, cls_b)
  return out


# speedup vs baseline: 1.0667x; 1.0120x over previous
import functools

import jax
import jax.numpy as jnp
from jax import lax
from jax.experimental import pallas as pl
from jax.experimental.pallas import tpu as pltpu
from jax.experimental.pallas import tpu_sc as plsc

_N = 10000
_E = 320000
_IN_DIM = 128
_H = 64
_NGRAPH = 128
_NCLS = 10

_NC = 2
_NS = 16
_L = 16
_NW = _NC * _NS

_CH = 80
_NCHUNK = 128
_EPW = _CH * _NCHUNK
_EPAD = _NW * _EPW
_NPAD = 10240
_RPT = _N // _NS
_ZPT = _NPAD // _NS
_ZB = 40
_NBUF = 8
_G = 6
_SLOW = 0
_GAT_NF, _GAT_NS = 144, 112
_AGG_NF, _AGG_NS = 200, 56

_mesh = plsc.VectorSubcoreMesh(
    core_axis_name="c", subcore_axis_name="s", num_cores=_NC, num_subcores=_NS)


def _zero_fill(buf, words_div16):
  z = jnp.zeros((_L,), jnp.float32)

  @pl.loop(0, words_div16)
  def _(i):
    buf[pl.ds(i * _L, _L)] = z


def _zero_fill2d(buf, nrows, ncols):
  z = jnp.zeros((_L,), jnp.float32)

  @pl.loop(0, nrows)
  def _(r):
    for q in range(ncols // _L):
      buf[r, pl.ds(q * _L, _L)] = z


def _sc_gat_body(src3, dst3, asrc_hbm, adst_hbm, xl_hbm, part_hbm, esum_hbm,
                 *scr):
  nck = jnp.where(lax.axis_index("c") == _SLOW, _GAT_NS, _GAT_NF)
  sidx_v, didx_v, asrc_v, adst_v, zbuf_v, zvec_v = scr[:6]
  bufs = scr[6:6 + _NBUF]
  exb = scr[6 + _NBUF:6 + 2 * _NBUF]
  acc_sh, esum_sh = scr[6 + 2 * _NBUF:8 + 2 * _NBUF]
  sems = scr[8 + 2 * _NBUF:]
  gsems = sems[:_NBUF]
  ssems = sems[_NBUF:2 * _NBUF]
  esems = sems[2 * _NBUF:3 * _NBUF]

  c = lax.axis_index("c")
  s = lax.axis_index("s")
  wid = s * _NC + c

  _zero_fill2d(zbuf_v, _ZB, _H)
  _zero_fill(zvec_v, _ZPT // _L)
  for i in range(_ZPT // _ZB):
    pltpu.async_copy(zbuf_v, acc_sh.at[pl.ds(s * _ZPT + i * _ZB, _ZB)],
                     ssems[i % _NBUF])
  pltpu.async_copy(zvec_v, esum_sh.at[pl.ds(s * _ZPT, _ZPT)], esems[0])
  pltpu.async_copy(src3.at[wid], sidx_v, gsems[0])
  pltpu.async_copy(dst3.at[wid], didx_v, gsems[1])
  pltpu.async_copy(asrc_hbm, asrc_v, gsems[2])
  pltpu.async_copy(adst_hbm, adst_v, gsems[3])
  for i in range(_ZPT // _ZB):
    pltpu.make_async_copy(zbuf_v, acc_sh.at[pl.ds(s * _ZPT, _ZB)],
                          ssems[i % _NBUF]).wait()
  pltpu.make_async_copy(zvec_v, esum_sh.at[pl.ds(s * _ZPT, _ZPT)],
                        esems[0]).wait()
  pltpu.make_async_copy(src3.at[wid], sidx_v, gsems[0]).wait()
  pltpu.make_async_copy(dst3.at[wid], didx_v, gsems[1]).wait()
  pltpu.make_async_copy(asrc_hbm, asrc_v, gsems[2]).wait()
  pltpu.make_async_copy(adst_hbm, adst_v, gsems[3]).wait()

  plsc.subcore_barrier()

  for i in range(_G):
    pltpu.async_copy(xl_hbm.at[sidx_v.at[i]], bufs[i], gsems[i])

  @pl.loop(0, nck, step=_NBUF)
  def _(ch):
    for b in range(_NBUF):
      cur = ch + b
      rb, eb = bufs[b], exb[b]
      nb = (b + _G) % _NBUF
      pltpu.make_async_copy(xl_hbm.at[sidx_v.at[cur]], rb, gsems[b]).wait()

      @pl.when(cur >= _NBUF - _G)
      def _():
        pltpu.make_async_copy(
            bufs[nb], acc_sh.at[didx_v.at[cur + _G - _NBUF]], ssems[nb]).wait()
        pltpu.make_async_copy(
            exb[nb], esum_sh.at[didx_v.at[cur + _G - _NBUF]], esems[nb]).wait()

      @pl.when(cur + _G < nck)
      def _():
        pltpu.async_copy(xl_hbm.at[sidx_v.at[cur + _G]], bufs[nb], gsems[nb])

      for j in range(_CH // _L):
        si = sidx_v[cur, pl.ds(j * _L, _L)]
        di = didx_v[cur, pl.ds(j * _L, _L)]
        e = plsc.load_gather(asrc_v, [si]) + plsc.load_gather(adst_v, [di])
        e = jnp.where(e > 0, e, 0.2 * e)
        eb[pl.ds(j * _L, _L)] = jnp.exp(e)

      @pl.loop(0, _CH // _L)
      def _(j):
        ex16 = eb[pl.ds(j * _L, _L)]
        for k in range(_L):
          r = j * _L + k
          w = ex16[k]
          for q in range(_H // _L):
            rb[r, pl.ds(q * _L, _L)] = rb[r, pl.ds(q * _L, _L)] * w

      pltpu.async_copy(rb, acc_sh.at[didx_v.at[cur]], ssems[b], add=True)
      pltpu.async_copy(eb, esum_sh.at[didx_v.at[cur]], esems[b], add=True)

  for dj in range(_NBUF - _G, 0, -1):
    bj = (_NBUF - dj) % _NBUF
    pltpu.make_async_copy(bufs[bj], acc_sh.at[didx_v.at[nck - dj]], ssems[bj]).wait()
    pltpu.make_async_copy(exb[bj], esum_sh.at[didx_v.at[nck - dj]], esems[bj]).wait()

  plsc.subcore_barrier()

  pltpu.sync_copy(acc_sh.at[pl.ds(s * _ZPT, _ZPT)],
                  part_hbm.at[c, pl.ds(s * _ZPT, _ZPT)])
  pltpu.sync_copy(esum_sh.at[pl.ds(s * _ZPT, _ZPT)],
                  esum_hbm.at[c, pl.ds(s * _ZPT, _ZPT)])


_sc_gat = functools.partial(
    pl.kernel,
    out_type=(
        jax.ShapeDtypeStruct((_NC, _NPAD, _H), jnp.float32),
        jax.ShapeDtypeStruct((_NC, _NPAD), jnp.float32),
    ),
    mesh=_mesh,
    compiler_params=pltpu.CompilerParams(
        needs_layout_passes=False, use_tc_tiling_on_sc=False),
    scratch_types=[
        pltpu.VMEM((_GAT_NF, _CH), jnp.int32),
        pltpu.VMEM((_GAT_NF, _CH), jnp.int32),
        pltpu.VMEM((_N,), jnp.float32),
        pltpu.VMEM((_N,), jnp.float32),
        pltpu.VMEM((_ZB, _H), jnp.float32),
        pltpu.VMEM((_ZPT,), jnp.float32),
    ] + [pltpu.VMEM((_CH, _H), jnp.float32) for _ in range(_NBUF)]
    + [pltpu.VMEM((_CH,), jnp.float32) for _ in range(_NBUF)]
    + [
        pltpu.VMEM_SHARED((_NPAD, _H), jnp.float32),
        pltpu.VMEM_SHARED((_NPAD,), jnp.float32),
    ] + [pltpu.SemaphoreType.DMA for _ in range(3 * _NBUF)],
)(_sc_gat_body)


def _sc_agg_body(src3, dst3, h_hbm, part_hbm, *scr):
  nck = jnp.where(lax.axis_index("c") == _SLOW, _AGG_NS, _AGG_NF)
  sidx_v, didx_v, zbuf_v = scr[:3]
  bufs = scr[3:3 + _NBUF]
  acc_sh = scr[3 + _NBUF]
  sems = scr[4 + _NBUF:]
  gsems = sems[:_NBUF]
  ssems = sems[_NBUF:2 * _NBUF]

  c = lax.axis_index("c")
  s = lax.axis_index("s")
  wid = s * _NC + c

  _zero_fill2d(zbuf_v, _ZB, _H)
  for i in range(_ZPT // _ZB):
    pltpu.async_copy(zbuf_v, acc_sh.at[pl.ds(s * _ZPT + i * _ZB, _ZB)],
                     ssems[i % _NBUF])
  pltpu.async_copy(src3.at[wid], sidx_v, gsems[0])
  pltpu.async_copy(dst3.at[wid], didx_v, gsems[1])
  for i in range(_ZPT // _ZB):
    pltpu.make_async_copy(zbuf_v, acc_sh.at[pl.ds(s * _ZPT, _ZB)],
                          ssems[i % _NBUF]).wait()
  pltpu.make_async_copy(src3.at[wid], sidx_v, gsems[0]).wait()
  pltpu.make_async_copy(dst3.at[wid], didx_v, gsems[1]).wait()

  plsc.subcore_barrier()

  for i in range(_G):
    pltpu.async_copy(h_hbm.at[sidx_v.at[i]], bufs[i], gsems[i])

  @pl.loop(0, nck, step=_NBUF)
  def _(ch):
    for b in range(_NBUF):
      cur = ch + b
      rb = bufs[b]
      nb = (b + _G) % _NBUF
      pltpu.make_async_copy(h_hbm.at[sidx_v.at[cur]], rb, gsems[b]).wait()

      @pl.when(cur >= _NBUF - _G)
      def _():
        pltpu.make_async_copy(
            bufs[nb], acc_sh.at[didx_v.at[cur + _G - _NBUF]], ssems[nb]).wait()

      @pl.when(cur + _G < nck)
      def _():
        pltpu.async_copy(h_hbm.at[sidx_v.at[cur + _G]], bufs[nb], gsems[nb])

      pltpu.async_copy(rb, acc_sh.at[didx_v.at[cur]], ssems[b], add=True)

  for dj in range(_NBUF - _G, 0, -1):
    bj = (_NBUF - dj) % _NBUF
    pltpu.make_async_copy(bufs[bj], acc_sh.at[didx_v.at[nck - dj]], ssems[bj]).wait()

  plsc.subcore_barrier()

  pltpu.sync_copy(acc_sh.at[pl.ds(s * _ZPT, _ZPT)],
                  part_hbm.at[c, pl.ds(s * _ZPT, _ZPT)])


_sc_agg = functools.partial(
    pl.kernel,
    out_type=jax.ShapeDtypeStruct((_NC, _NPAD, _H), jnp.float32),
    mesh=_mesh,
    compiler_params=pltpu.CompilerParams(
        needs_layout_passes=False, use_tc_tiling_on_sc=False),
    scratch_types=[
        pltpu.VMEM((_AGG_NF, _CH), jnp.int32),
        pltpu.VMEM((_AGG_NF, _CH), jnp.int32),
        pltpu.VMEM((_ZB, _H), jnp.float32),
    ] + [pltpu.VMEM((_CH, _H), jnp.float32) for _ in range(_NBUF)]
    + [pltpu.VMEM_SHARED((_NPAD, _H), jnp.float32)]
    + [pltpu.SemaphoreType.DMA for _ in range(2 * _NBUF)],
)(_sc_agg_body)


def _bn_relu(z, g, b):
  m = jnp.mean(z, axis=0)
  v = jnp.mean((z - m) ** 2, axis=0)
  return jnp.maximum((z - m) / jnp.sqrt(v + 1e-5) * g + b, 0.0)


def _tc_pre_body(x_ref, w_ref, as_ref, ad_ref, xl_ref, a2_ref):
  xl = jnp.dot(x_ref[...], w_ref[...], preferred_element_type=jnp.float32)
  xl_ref[...] = xl
  asrc = jnp.sum(xl * as_ref[...][None, :], axis=1)
  adst = jnp.sum(xl * ad_ref[...][None, :], axis=1)
  a2_ref[...] = jnp.concatenate([asrc[:, None], adst[:, None]], axis=1)


def _tc_gat_post_body(part_ref, esum_ref, gb_ref, g_ref, b_ref, out_ref):
  p = part_ref[0, :_N] + part_ref[1, :_N]
  es = esum_ref[0, :_N] + esum_ref[1, :_N]
  z = p / (es[:, None] + 1e-16) + gb_ref[...][None, :]
  out_ref[...] = _bn_relu(z, g_ref[...][None, :], b_ref[...][None, :])


def _tc_gin_body(h_ref, part_ref, w1_ref, b1_ref, ng_ref, nb_ref,
                 w2_ref, b2_ref, og_ref, ob_ref, out_ref):
  z = h_ref[...] + part_ref[0, :_N] + part_ref[1, :_N]
  t = jnp.dot(z, w1_ref[...], preferred_element_type=jnp.float32)
  t = _bn_relu(t + b1_ref[...][None, :], ng_ref[...][None, :],
               nb_ref[...][None, :])
  t = jnp.dot(t, w2_ref[...], preferred_element_type=jnp.float32)
  t = t + b2_ref[...][None, :]
  out_ref[...] = _bn_relu(t, og_ref[...][None, :], ob_ref[...][None, :])


def _tc_final_body(g1_ref, g2_ref, g3_ref, batch_ref,
                   f1w_ref, f1b_ref, f2w_ref, f2b_ref, cw_ref, cb_ref,
                   out_ref):
  gcn = jnp.concatenate([g1_ref[...], g2_ref[...], g3_ref[...]], axis=1)
  gid = lax.broadcasted_iota(jnp.int32, (_NGRAPH, 1), 0)
  onehot = (gid == batch_ref[...]).astype(jnp.float32)
  pooled = jnp.dot(onehot, gcn, preferred_element_type=jnp.float32)
  h = jnp.dot(pooled, f1w_ref[...], preferred_element_type=jnp.float32)
  h = h + f1b_ref[...][None, :]
  h = jnp.dot(h, f2w_ref[...], preferred_element_type=jnp.float32)
  h = jnp.maximum(h + f2b_ref[...][None, :], 0.0)
  o = jnp.dot(h, cw_ref[...], preferred_element_type=jnp.float32)
  out_ref[...] = o + cb_ref[...][None, :]


def kernel(x, edge, batch, gat_W, gat_att_src, gat_att_dst, gat_b, bn1_g,
           bn1_b, g2_W1, g2_b1, g2_ng, g2_nb, g2_W2, g2_b2, g2_og, g2_ob,
           g3_W1, g3_b1, g3_ng, g3_nb, g3_W2, g3_b2, g3_og, g3_ob,
           fc1_W, fc1_b, fc2_W, fc2_b, cls_W, cls_b):
  src = edge[0].astype(jnp.int32)
  dst = edge[1].astype(jnp.int32)
  pad = _EPAD - _E
  src_p = jnp.concatenate([src, jnp.zeros((pad,), jnp.int32)])
  dst_p = jnp.concatenate([dst, jnp.full((pad,), _N, jnp.int32)])

  def _layout(x, n_fast, n_slow, fill):
    cut = _NS * n_fast * _CH
    e_fast = x[:cut].reshape(_NS, n_fast, _CH)
    e_slow = jnp.pad(
        x[cut:].reshape(_NS, n_slow, _CH),
        ((0, 0), (0, n_fast - n_slow), (0, 0)), constant_values=fill)
    pair = (e_slow, e_fast) if _SLOW == 0 else (e_fast, e_slow)
    return jnp.stack(pair, axis=1).reshape(_NW, n_fast, _CH)

  src3g = _layout(src_p, _GAT_NF, _GAT_NS, 0)
  dst3g = _layout(dst_p, _GAT_NF, _GAT_NS, _N)
  src3a = _layout(src_p, _AGG_NF, _AGG_NS, 0)
  dst3a = _layout(dst_p, _AGG_NF, _AGG_NS, _N)
  batch_row = batch.astype(jnp.int32)[None, :]

  xl, a2 = pl.pallas_call(
      _tc_pre_body,
      out_shape=(
          jax.ShapeDtypeStruct((_N, _H), jnp.float32),
          jax.ShapeDtypeStruct((_N, 2), jnp.float32),
      ),
  )(x, gat_W, gat_att_src, gat_att_dst)
  asrc = a2[:, 0]
  adst = a2[:, 1]

  part1, esum = _sc_gat(src3g, dst3g, asrc, adst, xl)

  gcn1 = pl.pallas_call(
      _tc_gat_post_body,
      out_shape=jax.ShapeDtypeStruct((_N, _H), jnp.float32),
  )(part1, esum, gat_b, bn1_g, bn1_b)

  part2 = _sc_agg(src3a, dst3a, gcn1)
  gcn2 = pl.pallas_call(
      _tc_gin_body,
      out_shape=jax.ShapeDtypeStruct((_N, _H), jnp.float32),
  )(gcn1, part2, g2_W1, g2_b1, g2_ng, g2_nb, g2_W2, g2_b2, g2_og, g2_ob)

  part3 = _sc_agg(src3a, dst3a, gcn2)
  gcn3 = pl.pallas_call(
      _tc_gin_body,
      out_shape=jax.ShapeDtypeStruct((_N, _H), jnp.float32),
  )(gcn2, part3, g3_W1, g3_b1, g3_ng, g3_nb, g3_W2, g3_b2, g3_og, g3_ob)

  out = pl.pallas_call(
      _tc_final_body,
      out_shape=jax.ShapeDtypeStruct((_NGRAPH, _NCLS), jnp.float32),
  )(gcn1, gcn2, gcn3, batch_row, fc1_W, fc1_b, fc2_W, fc2_b, cls_W, cls_b)
  return out
